# Initial kernel scaffold; baseline (speedup 1.0000x reference)
import jax
import jax.numpy as jnp
from jax.experimental import pallas as pl


def kernel(h, edge_index, e, W_emb, b_emb, W_ggc, W_ih, W_hh, b_ih, b_hh, gamma, beta, W_mlp, b_mlp):
    raise NotImplementedError("write your pallas kernel here")


# speedup vs baseline: 1.0026x; 1.0026x over previous
import functools

import jax
import jax.numpy as jnp
from jax.experimental import pallas as pl
from jax.experimental.pallas import tpu as pltpu

N_NODES = 10000
HID = 256
ROW_BLK = 1000


def _emb_body(h_ref, w_ref, b_ref, o_ref):
    o_ref[...] = jnp.dot(h_ref[...], w_ref[...],
                         preferred_element_type=jnp.float32) + b_ref[...]


def _dense1_body(x_ref, wg_ref, whh_ref, bhh_ref, m_ref, gh_ref):
    x = x_ref[...]
    m_ref[...] = jnp.dot(x, wg_ref[...], preferred_element_type=jnp.float32)
    gh_ref[...] = jnp.dot(x, whh_ref[...],
                          preferred_element_type=jnp.float32) + bhh_ref[...]


def _gru_body(agg_ref, gh_ref, x_ref, wih_ref, bih_ref, o_ref):
    agg = agg_ref[...]
    gi = jnp.dot(agg, wih_ref[...],
                 preferred_element_type=jnp.float32) + bih_ref[...]
    gh = gh_ref[...]
    x = x_ref[...]
    i_r = gi[:, :HID]
    i_z = gi[:, HID:2 * HID]
    i_n = gi[:, 2 * HID:]
    h_r = gh[:, :HID]
    h_z = gh[:, HID:2 * HID]
    h_n = gh[:, 2 * HID:]
    r = jax.nn.sigmoid(i_r + h_r)
    z = jax.nn.sigmoid(i_z + h_z)
    n = jnp.tanh(i_n + r * h_n)
    o_ref[...] = (1.0 - z) * n + z * x


def _bn_mlp_body(x_ref, hin_ref, gamma_ref, beta_ref, wmlp_ref, bmlp_ref,
                 o_ref):
    x = x_ref[...]
    mean = jnp.mean(x, axis=0, keepdims=True)
    var = jnp.mean((x - mean) ** 2, axis=0, keepdims=True)
    xn = (x - mean) * jax.lax.rsqrt(var + 1e-5) * gamma_ref[...] + beta_ref[...]
    y = hin_ref[...] + xn
    o_ref[...] = jnp.dot(y, wmlp_ref[...],
                         preferred_element_type=jnp.float32) + bmlp_ref[...]


def _emb(h, W_emb, b_emb):
    n_blk = N_NODES // ROW_BLK
    return pl.pallas_call(
        _emb_body,
        grid=(n_blk,),
        in_specs=[
            pl.BlockSpec((ROW_BLK, 128), lambda i: (i, 0)),
            pl.BlockSpec((128, HID), lambda i: (0, 0)),
            pl.BlockSpec((1, HID), lambda i: (0, 0)),
        ],
        out_specs=pl.BlockSpec((ROW_BLK, HID), lambda i: (i, 0)),
        out_shape=jax.ShapeDtypeStruct((N_NODES, HID), jnp.float32),
    )(h, W_emb, b_emb)


def _dense1(x, W_g, W_hhT, b_hh):
    n_blk = N_NODES // ROW_BLK
    return pl.pallas_call(
        _dense1_body,
        grid=(n_blk,),
        in_specs=[
            pl.BlockSpec((ROW_BLK, HID), lambda i: (i, 0)),
            pl.BlockSpec((HID, HID), lambda i: (0, 0)),
            pl.BlockSpec((HID, 3 * HID), lambda i: (0, 0)),
            pl.BlockSpec((1, 3 * HID), lambda i: (0, 0)),
        ],
        out_specs=[
            pl.BlockSpec((ROW_BLK, HID), lambda i: (i, 0)),
            pl.BlockSpec((ROW_BLK, 3 * HID), lambda i: (i, 0)),
        ],
        out_shape=[
            jax.ShapeDtypeStruct((N_NODES, HID), jnp.float32),
            jax.ShapeDtypeStruct((N_NODES, 3 * HID), jnp.float32),
        ],
    )(x, W_g, W_hhT, b_hh)


def _gru(agg, gh, x, W_ihT, b_ih):
    n_blk = N_NODES // ROW_BLK
    return pl.pallas_call(
        _gru_body,
        grid=(n_blk,),
        in_specs=[
            pl.BlockSpec((ROW_BLK, HID), lambda i: (i, 0)),
            pl.BlockSpec((ROW_BLK, 3 * HID), lambda i: (i, 0)),
            pl.BlockSpec((ROW_BLK, HID), lambda i: (i, 0)),
            pl.BlockSpec((HID, 3 * HID), lambda i: (0, 0)),
            pl.BlockSpec((1, 3 * HID), lambda i: (0, 0)),
        ],
        out_specs=pl.BlockSpec((ROW_BLK, HID), lambda i: (i, 0)),
        out_shape=jax.ShapeDtypeStruct((N_NODES, HID), jnp.float32),
    )(agg, gh, x, W_ihT, b_ih)


def _bn_mlp(x, h_in, gamma, beta, W_mlp, b_mlp):
    return pl.pallas_call(
        _bn_mlp_body,
        out_shape=jax.ShapeDtypeStruct((N_NODES, 40), jnp.float32),
    )(x, h_in, gamma.reshape(1, HID), beta.reshape(1, HID), W_mlp,
      b_mlp.reshape(1, 40))


def kernel(h, edge_index, e, W_emb, b_emb, W_ggc, W_ih, W_hh, b_ih, b_hh,
           gamma, beta, W_mlp, b_mlp):
    src = edge_index[0]
    dst = edge_index[1]
    W_ihT = W_ih.T
    W_hhT = W_hh.T
    b_ih2 = b_ih.reshape(1, 3 * HID)
    b_hh2 = b_hh.reshape(1, 3 * HID)
    b_emb2 = b_emb.reshape(1, HID)

    x = _emb(h, W_emb, b_emb2)
    h_in = x
    for i in range(3):
        m, gh = _dense1(x, W_ggc[i], W_hhT, b_hh2)
        msg = e[:, None] * jnp.take(m, src, axis=0)
        agg = jnp.zeros_like(x).at[dst].add(msg)
        x = _gru(agg, gh, x, W_ihT, b_ih2)
    return _bn_mlp(x, h_in, gamma, beta, W_mlp, b_mlp)


# speedup vs baseline: 5.1290x; 5.1157x over previous
import functools

import jax
import jax.numpy as jnp
from jax import lax
from jax.experimental import pallas as pl
from jax.experimental.pallas import tpu as pltpu
from jax.experimental.pallas import tpu_sc as plsc

N_NODES = 10000
N_EDGES = 320000
HID = 256
HALF = 128
ROW_BLK = 1000

NC = 2
NS = 16
LANES = 16

E_PER_TILE = N_EDGES // NS
CHUNK = 80
N_CHUNK = E_PER_TILE // CHUNK
ROWS_PER_TILE = 624


def _emb_body(h_ref, w_ref, b_ref, o_ref):
    o_ref[...] = jnp.dot(h_ref[...], w_ref[...],
                         preferred_element_type=jnp.float32) + b_ref[...]


def _emb(h, W_emb, b_emb):
    n_blk = N_NODES // ROW_BLK
    return pl.pallas_call(
        _emb_body,
        grid=(n_blk,),
        in_specs=[
            pl.BlockSpec((ROW_BLK, 128), lambda i: (i, 0)),
            pl.BlockSpec((128, HID), lambda i: (0, 0)),
            pl.BlockSpec((1, HID), lambda i: (0, 0)),
        ],
        out_specs=pl.BlockSpec((ROW_BLK, HID), lambda i: (i, 0)),
        out_shape=jax.ShapeDtypeStruct((N_NODES, HID), jnp.float32),
    )(h, W_emb, b_emb)


def _mm_g_body(x_ref, wg_ref, lo_ref, hi_ref):
    m = jnp.dot(x_ref[...], wg_ref[...], preferred_element_type=jnp.float32)
    lo_ref[...] = m[:, :HALF]
    hi_ref[...] = m[:, HALF:]


def _mm_g(x, W_g):
    n_blk = N_NODES // ROW_BLK
    return pl.pallas_call(
        _mm_g_body,
        grid=(n_blk,),
        in_specs=[
            pl.BlockSpec((ROW_BLK, HID), lambda i: (i, 0)),
            pl.BlockSpec((HID, HID), lambda i: (0, 0)),
        ],
        out_specs=[
            pl.BlockSpec((ROW_BLK, HALF), lambda i: (i, 0)),
            pl.BlockSpec((ROW_BLK, HALF), lambda i: (i, 0)),
        ],
        out_shape=[
            jax.ShapeDtypeStruct((N_NODES, HALF), jnp.float32),
            jax.ShapeDtypeStruct((N_NODES, HALF), jnp.float32),
        ],
    )(x, W_g)


def _mm_hh_body(x_ref, whh_ref, bhh_ref, gh_ref):
    gh_ref[...] = jnp.dot(x_ref[...], whh_ref[...],
                          preferred_element_type=jnp.float32) + bhh_ref[...]


def _mm_hh(x, W_hhT, b_hh):
    n_blk = N_NODES // ROW_BLK
    return pl.pallas_call(
        _mm_hh_body,
        grid=(n_blk,),
        in_specs=[
            pl.BlockSpec((ROW_BLK, HID), lambda i: (i, 0)),
            pl.BlockSpec((HID, 3 * HID), lambda i: (0, 0)),
            pl.BlockSpec((1, 3 * HID), lambda i: (0, 0)),
        ],
        out_specs=pl.BlockSpec((ROW_BLK, 3 * HID), lambda i: (i, 0)),
        out_shape=jax.ShapeDtypeStruct((N_NODES, 3 * HID), jnp.float32),
    )(x, W_hhT, b_hh)


def _gru_body(lo_ref, hi_ref, gh_ref, x_ref, wih_ref, bih_ref, o_ref):
    gi = (jnp.dot(lo_ref[...], wih_ref[:HALF, :],
                  preferred_element_type=jnp.float32)
          + jnp.dot(hi_ref[...], wih_ref[HALF:, :],
                    preferred_element_type=jnp.float32)
          + bih_ref[...])
    gh = gh_ref[...]
    x = x_ref[...]
    i_r = gi[:, :HID]
    i_z = gi[:, HID:2 * HID]
    i_n = gi[:, 2 * HID:]
    h_r = gh[:, :HID]
    h_z = gh[:, HID:2 * HID]
    h_n = gh[:, 2 * HID:]
    r = jax.nn.sigmoid(i_r + h_r)
    z = jax.nn.sigmoid(i_z + h_z)
    n = jnp.tanh(i_n + r * h_n)
    o_ref[...] = (1.0 - z) * n + z * x


def _gru(agg_lo, agg_hi, gh, x, W_ihT, b_ih):
    n_blk = N_NODES // ROW_BLK
    return pl.pallas_call(
        _gru_body,
        grid=(n_blk,),
        in_specs=[
            pl.BlockSpec((ROW_BLK, HALF), lambda i: (i, 0)),
            pl.BlockSpec((ROW_BLK, HALF), lambda i: (i, 0)),
            pl.BlockSpec((ROW_BLK, 3 * HID), lambda i: (i, 0)),
            pl.BlockSpec((ROW_BLK, HID), lambda i: (i, 0)),
            pl.BlockSpec((HID, 3 * HID), lambda i: (0, 0)),
            pl.BlockSpec((1, 3 * HID), lambda i: (0, 0)),
        ],
        out_specs=pl.BlockSpec((ROW_BLK, HID), lambda i: (i, 0)),
        out_shape=jax.ShapeDtypeStruct((N_NODES, HID), jnp.float32),
    )(agg_lo, agg_hi, gh, x, W_ihT, b_ih)


def _bn_mlp_body(x_ref, hin_ref, gamma_ref, beta_ref, wmlp_ref, bmlp_ref,
                 o_ref):
    x = x_ref[...]
    mean = jnp.mean(x, axis=0, keepdims=True)
    var = jnp.mean((x - mean) ** 2, axis=0, keepdims=True)
    xn = (x - mean) * lax.rsqrt(var + 1e-5) * gamma_ref[...] + beta_ref[...]
    y = hin_ref[...] + xn
    o_ref[...] = jnp.dot(y, wmlp_ref[...],
                         preferred_element_type=jnp.float32) + bmlp_ref[...]


def _bn_mlp(x, h_in, gamma, beta, W_mlp, b_mlp):
    return pl.pallas_call(
        _bn_mlp_body,
        out_shape=jax.ShapeDtypeStruct((N_NODES, 40), jnp.float32),
    )(x, h_in, gamma.reshape(1, HID), beta.reshape(1, HID), W_mlp,
      b_mlp.reshape(1, 40))


def _scale_chunk(rows, evals, n):
    def body(k, _):
        ev16 = evals[pl.ds(k * LANES, LANES)]
        for l in range(LANES):
            j = k * LANES + l
            sv = jnp.full((LANES,), ev16[l], jnp.float32)
            for v in range(HALF // LANES):
                sl = pl.ds(v * LANES, LANES)
                rows[j, sl] = rows[j, sl] * sv
        return 0
    lax.fori_loop(0, n // LANES, body, 0)


def _agg_body(mlo, mhi, src, dst, ew, out_lo, out_hi, acc,
              rows0, rows1, si0, si1, di0, di1, ev0, ev1, sd0, sd1, zb,
              sem_i0, sem_i1, sem_g0, sem_g1, sem_s0, sem_s1):
    c = lax.axis_index("c")
    s = lax.axis_index("s")
    tbase = s * E_PER_TILE

    rows = (rows0, rows1)
    si = (si0, si1)
    di = (di0, di1)
    ev = (ev0, ev1)
    sd = (sd0, sd1)
    sem_i = (sem_i0, sem_i1)
    sem_g = (sem_g0, sem_g1)
    sem_s = (sem_s0, sem_s1)

    def zrow(j, _):
        zv = jnp.zeros((LANES,), jnp.float32)
        for v in range(HALF // LANES):
            zb[j, pl.ds(v * LANES, LANES)] = zv
        return 0
    lax.fori_loop(0, CHUNK, zrow, 0, unroll=4)
    rbase = s * ROWS_PER_TILE

    @pl.when(s != NS - 1)
    def _():
        for k in range(7):
            pltpu.sync_copy(zb, acc.at[pl.ds(rbase + k * CHUNK, CHUNK)])
        pltpu.sync_copy(zb.at[pl.ds(0, 64)],
                        acc.at[pl.ds(rbase + 560, 64)])

    @pl.when(s == NS - 1)
    def _():
        for k in range(8):
            pltpu.sync_copy(zb, acc.at[pl.ds(rbase + k * CHUNK, CHUNK)])

    plsc.subcore_barrier()

    def issue_idx(i, b):
        off = pl.ds(tbase + i * CHUNK, CHUNK)
        pltpu.async_copy(src.at[off], si[b], sem_i[b])
        pltpu.async_copy(dst.at[off], di[b], sem_i[b])
        pltpu.async_copy(ew.at[off], ev[b], sem_i[b])

    def wait_idx(b):
        pltpu.make_async_copy(src.at[pl.ds(0, CHUNK)], si[b], sem_i[b]).wait()
        pltpu.make_async_copy(dst.at[pl.ds(0, CHUNK)], di[b], sem_i[b]).wait()
        pltpu.make_async_copy(ew.at[pl.ds(0, CHUNK)], ev[b], sem_i[b]).wait()

    def issue_gather(b):
        @pl.when(c == 0)
        def _():
            pltpu.async_copy(mlo.at[si[b]], rows[b], sem_g[b])

        @pl.when(c != 0)
        def _():
            pltpu.async_copy(mhi.at[si[b]], rows[b], sem_g[b])

    def wait_gather(b):
        pltpu.make_async_copy(mlo.at[si[b]], rows[b], sem_g[b]).wait()

    def copy_dst(b):
        for v in range(CHUNK // LANES):
            sl = pl.ds(v * LANES, LANES)
            sd[b][sl] = di[b][sl]

    def issue_scatter(b):
        pltpu.async_copy(rows[b], acc.at[sd[b]], sem_s[b], add=True)

    def wait_scatter(b):
        pltpu.make_async_copy(rows[b], acc.at[sd[b]], sem_s[b]).wait()

    issue_idx(0, 0)
    wait_idx(0)
    issue_gather(0)
    issue_idx(1, 1)

    def step(it, _):
        i0 = it * 2
        for b in range(2):
            i = i0 + b
            nb = 1 - b

            wait_gather(b)
            _scale_chunk(rows[b], ev[b], CHUNK)
            copy_dst(b)
            issue_scatter(b)

            @pl.when(i + 2 < N_CHUNK)
            def _():
                issue_idx(i + 2, b)

            @pl.when(i + 1 < N_CHUNK)
            def _():
                @pl.when(i >= 1)
                def _():
                    wait_scatter(nb)

                wait_idx(nb)
                issue_gather(nb)
        return 0

    lax.fori_loop(0, N_CHUNK // 2, step, 0)

    wait_scatter(0)
    wait_scatter(1)
    plsc.subcore_barrier()

    for last, nrows in ((False, ROWS_PER_TILE), (True, 640)):
        osl = pl.ds(rbase, nrows)

        @pl.when((c == 0) & ((s == NS - 1) == last))
        def _():
            pltpu.sync_copy(acc.at[osl], out_lo.at[osl])

        @pl.when((c != 0) & ((s == NS - 1) == last))
        def _():
            pltpu.sync_copy(acc.at[osl], out_hi.at[osl])


_sc_aggregate = pl.kernel(
    _agg_body,
    out_type=[
        jax.ShapeDtypeStruct((N_NODES, HALF), jnp.float32),
        jax.ShapeDtypeStruct((N_NODES, HALF), jnp.float32),
    ],
    mesh=plsc.VectorSubcoreMesh(core_axis_name="c", subcore_axis_name="s",
                                num_cores=NC, num_subcores=NS),
    scratch_types=[
        pltpu.VMEM_SHARED((N_NODES, HALF), jnp.float32),
        pltpu.VMEM((CHUNK, HALF), jnp.float32),
        pltpu.VMEM((CHUNK, HALF), jnp.float32),
        pltpu.VMEM((CHUNK,), jnp.int32),
        pltpu.VMEM((CHUNK,), jnp.int32),
        pltpu.VMEM((CHUNK,), jnp.int32),
        pltpu.VMEM((CHUNK,), jnp.int32),
        pltpu.VMEM((CHUNK,), jnp.float32),
        pltpu.VMEM((CHUNK,), jnp.float32),
        pltpu.VMEM((CHUNK,), jnp.int32),
        pltpu.VMEM((CHUNK,), jnp.int32),
        pltpu.VMEM((CHUNK, HALF), jnp.float32),
        pltpu.SemaphoreType.DMA,
        pltpu.SemaphoreType.DMA,
        pltpu.SemaphoreType.DMA,
        pltpu.SemaphoreType.DMA,
        pltpu.SemaphoreType.DMA,
        pltpu.SemaphoreType.DMA,
    ],
)


def kernel(h, edge_index, e, W_emb, b_emb, W_ggc, W_ih, W_hh, b_ih, b_hh,
           gamma, beta, W_mlp, b_mlp):
    src = edge_index[0].astype(jnp.int32)
    dst = edge_index[1].astype(jnp.int32)
    W_ihT = W_ih.T
    W_hhT = W_hh.T
    b_ih2 = b_ih.reshape(1, 3 * HID)
    b_hh2 = b_hh.reshape(1, 3 * HID)
    b_emb2 = b_emb.reshape(1, HID)

    x = _emb(h, W_emb, b_emb2)
    h_in = x
    for i in range(3):
        m_lo, m_hi = _mm_g(x, W_ggc[i])
        agg_lo, agg_hi = _sc_aggregate(m_lo, m_hi, src, dst, e)
        gh = _mm_hh(x, W_hhT, b_hh2)
        x = _gru(agg_lo, agg_hi, gh, x, W_ihT, b_ih2)
    return _bn_mlp(x, h_in, gamma, beta, W_mlp, b_mlp)
